# Initial kernel scaffold; baseline (speedup 1.0000x reference)
import jax
import jax.numpy as jnp
from jax.experimental import pallas as pl


def kernel(x_flat, w_stack, b_stack):
    raise NotImplementedError("write your pallas kernel here")


# speedup vs baseline: 1.4064x; 1.4064x over previous
import jax
import jax.numpy as jnp
from jax.experimental import pallas as pl
from jax.experimental.pallas import tpu as pltpu

_DIMS = [(512, 256), (256, 128), (128, 128), (128, 128), (128, 256), (256, 512)]
_RELU = (True, True, False, True, True, False)
_LATENT = 64
_BATCH_TILE = 1024


def _ae_kernel(x_ref, w0, w1, w2, w3, w4, w5, b0, b1, b2, b3, b4, b5,
               enc_ref, dec_ref):
    ws = (w0, w1, w2, w3, w4, w5)
    bs = (b0, b1, b2, b3, b4, b5)
    h = x_ref[...].astype(jnp.bfloat16)
    for l in range(6):
        h = jnp.dot(h, ws[l][...], preferred_element_type=jnp.float32) + bs[l][...]
        if _RELU[l]:
            h = jnp.maximum(h, 0.0)
        if l == 2:
            enc_ref[...] = h[:, :_LATENT]
        if l < 5:
            h = h.astype(jnp.bfloat16)
    dec_ref[...] = h


def kernel(x_flat, w_stack, b_stack):
    B, d_in = x_flat.shape

    ws = [w_stack[l, :din, :dout].astype(jnp.bfloat16)
          for l, (din, dout) in enumerate(_DIMS)]
    bs = [b_stack[l, :, :dout] for l, (_, dout) in enumerate(_DIMS)]

    tm = min(_BATCH_TILE, B)
    n_tiles = (B + tm - 1) // tm
    B_pad = n_tiles * tm
    if B_pad != B:
        x_flat = jnp.zeros((B_pad, d_in), x_flat.dtype).at[:B].set(x_flat)

    w_specs = [pl.BlockSpec((din, dout), lambda i: (0, 0))
               for (din, dout) in _DIMS]
    b_specs = [pl.BlockSpec((1, dout), lambda i: (0, 0))
               for (_, dout) in _DIMS]

    enc, dec = pl.pallas_call(
        _ae_kernel,
        out_shape=(jax.ShapeDtypeStruct((B_pad, _LATENT), jnp.float32),
                   jax.ShapeDtypeStruct((B_pad, d_in), jnp.float32)),
        grid=(n_tiles,),
        in_specs=[pl.BlockSpec((tm, d_in), lambda i: (i, 0))] + w_specs + b_specs,
        out_specs=(pl.BlockSpec((tm, _LATENT), lambda i: (i, 0)),
                   pl.BlockSpec((tm, d_in), lambda i: (i, 0))),
        compiler_params=pltpu.CompilerParams(
            dimension_semantics=("parallel",)),
    )(x_flat, *ws, *bs)

    if B_pad != B:
        enc, dec = enc[:B], dec[:B]
    return enc, dec


# speedup vs baseline: 1.5872x; 1.1285x over previous
import jax
import jax.numpy as jnp
from jax.experimental import pallas as pl
from jax.experimental.pallas import tpu as pltpu

_DIMS = [(512, 256), (256, 128), (128, 128), (128, 128), (128, 256), (256, 512)]
_RELU = (True, True, False, True, True, False)
_LATENT = 64
_BATCH_TILE = 2048


def _ae_kernel(x_ref, w0, w1, w2, w3, w4, w5, b0, b1, b2, b3, b4, b5,
               enc_ref, dec_ref):
    ws = (w0, w1, w2, w3, w4, w5)
    bs = (b0, b1, b2, b3, b4, b5)
    h = x_ref[...].astype(jnp.bfloat16)
    for l in range(6):
        h = jnp.dot(h, ws[l][...], preferred_element_type=jnp.float32) + bs[l][...]
        if _RELU[l]:
            h = jnp.maximum(h, 0.0)
        if l == 2:
            enc_ref[...] = h[:, :_LATENT]
        if l < 5:
            h = h.astype(jnp.bfloat16)
    dec_ref[...] = h


def kernel(x_flat, w_stack, b_stack):
    B, d_in = x_flat.shape

    ws = [w_stack[l, :din, :dout].astype(jnp.bfloat16)
          for l, (din, dout) in enumerate(_DIMS)]
    bs = [b_stack[l, :, :dout] for l, (_, dout) in enumerate(_DIMS)]

    tm = min(_BATCH_TILE, B)
    n_tiles = (B + tm - 1) // tm
    B_pad = n_tiles * tm
    if B_pad != B:
        x_flat = jnp.zeros((B_pad, d_in), x_flat.dtype).at[:B].set(x_flat)

    w_specs = [pl.BlockSpec((din, dout), lambda i: (0, 0))
               for (din, dout) in _DIMS]
    b_specs = [pl.BlockSpec((1, dout), lambda i: (0, 0))
               for (_, dout) in _DIMS]

    enc, dec = pl.pallas_call(
        _ae_kernel,
        out_shape=(jax.ShapeDtypeStruct((B_pad, _LATENT), jnp.float32),
                   jax.ShapeDtypeStruct((B_pad, d_in), jnp.float32)),
        grid=(n_tiles,),
        in_specs=[pl.BlockSpec((tm, d_in), lambda i: (i, 0))] + w_specs + b_specs,
        out_specs=(pl.BlockSpec((tm, _LATENT), lambda i: (i, 0)),
                   pl.BlockSpec((tm, d_in), lambda i: (i, 0))),
        compiler_params=pltpu.CompilerParams(
            dimension_semantics=("parallel",)),
    )(x_flat, *ws, *bs)

    if B_pad != B:
        enc, dec = enc[:B], dec[:B]
    return enc, dec


# speedup vs baseline: 1.6297x; 1.0268x over previous
import jax
import jax.numpy as jnp
from jax.experimental import pallas as pl
from jax.experimental.pallas import tpu as pltpu

_DIMS = [(512, 256), (256, 128), (128, 128), (128, 128), (128, 256), (256, 512)]
_RELU = (True, True, False, True, True, False)
_LATENT = 64
_BATCH_TILE = 4096


def _ae_kernel(x_ref, w0, w1, w2, w3, w4, w5, b0, b1, b2, b3, b4, b5,
               enc_ref, dec_ref):
    ws = (w0, w1, w2, w3, w4, w5)
    bs = (b0, b1, b2, b3, b4, b5)
    h = x_ref[...].astype(jnp.bfloat16)
    for l in range(6):
        h = jnp.dot(h, ws[l][...], preferred_element_type=jnp.float32) + bs[l][...]
        if _RELU[l]:
            h = jnp.maximum(h, 0.0)
        if l == 2:
            enc_ref[...] = h[:, :_LATENT]
        if l < 5:
            h = h.astype(jnp.bfloat16)
    dec_ref[...] = h


def kernel(x_flat, w_stack, b_stack):
    B, d_in = x_flat.shape

    ws = [w_stack[l, :din, :dout].astype(jnp.bfloat16)
          for l, (din, dout) in enumerate(_DIMS)]
    bs = [b_stack[l, :, :dout] for l, (_, dout) in enumerate(_DIMS)]

    tm = min(_BATCH_TILE, B)
    n_tiles = (B + tm - 1) // tm
    B_pad = n_tiles * tm
    if B_pad != B:
        x_flat = jnp.zeros((B_pad, d_in), x_flat.dtype).at[:B].set(x_flat)

    w_specs = [pl.BlockSpec((din, dout), lambda i: (0, 0))
               for (din, dout) in _DIMS]
    b_specs = [pl.BlockSpec((1, dout), lambda i: (0, 0))
               for (_, dout) in _DIMS]

    enc, dec = pl.pallas_call(
        _ae_kernel,
        out_shape=(jax.ShapeDtypeStruct((B_pad, _LATENT), jnp.float32),
                   jax.ShapeDtypeStruct((B_pad, d_in), jnp.float32)),
        grid=(n_tiles,),
        in_specs=[pl.BlockSpec((tm, d_in), lambda i: (i, 0))] + w_specs + b_specs,
        out_specs=(pl.BlockSpec((tm, _LATENT), lambda i: (i, 0)),
                   pl.BlockSpec((tm, d_in), lambda i: (i, 0))),
        compiler_params=pltpu.CompilerParams(
            dimension_semantics=("parallel",)),
    )(x_flat, *ws, *bs)

    if B_pad != B:
        enc, dec = enc[:B], dec[:B]
    return enc, dec
